# Initial kernel scaffold; baseline (speedup 1.0000x reference)
import jax
import jax.numpy as jnp
from jax.experimental import pallas as pl


def kernel(x_in, weight):
    raise NotImplementedError("write your pallas kernel here")


# speedup vs baseline: 2.9563x; 2.9563x over previous
import functools

import jax
import jax.numpy as jnp
from jax import lax
from jax.experimental import pallas as pl
from jax.experimental.pallas import tpu as pltpu
from jax.experimental.pallas import tpu_sc as plsc

_VOCAB = 100000
_DIM = 128
_BATCH = 4096
_SEQ = 50
_B = _BATCH * _SEQ
_NC = 2
_NS = 16
_NW = _NC * _NS
_BPW = _B // _NW
_CH = 128
_NCH = _BPW // _CH


def _make_emb_kernel():
    mesh = plsc.VectorSubcoreMesh(core_axis_name="c", subcore_axis_name="s")

    @functools.partial(
        pl.kernel,
        mesh=mesh,
        out_type=jax.ShapeDtypeStruct((_B, _DIM), jnp.float32),
        scratch_types=[
            pltpu.VMEM((_NCH, _CH), jnp.int32),
            pltpu.VMEM((_CH, _DIM), jnp.float32),
            pltpu.SemaphoreType.DMA,
        ],
    )
    def emb(idx_hbm, table_hbm, out_hbm, idx_v, buf, sem):
        wid = lax.axis_index("s") * _NC + lax.axis_index("c")
        base = wid * _BPW
        pltpu.sync_copy(idx_hbm.at[wid], idx_v)

        def body(j, carry):
            pltpu.async_copy(table_hbm.at[idx_v.at[j]], buf, sem).wait()
            pltpu.sync_copy(buf, out_hbm.at[pl.ds(base + j * _CH, _CH)])
            return carry

        lax.fori_loop(0, _NCH, body, 0)

    return emb


_emb = _make_emb_kernel()


def kernel(x_in, weight):
    idx = x_in.astype(jnp.int32).reshape(_NW, _NCH, _CH)
    out = _emb(idx, weight)
    return out.reshape(_BATCH, _SEQ, _DIM)


# speedup vs baseline: 3.3066x; 1.1185x over previous
import functools

import jax
import jax.numpy as jnp
from jax import lax
from jax.experimental import pallas as pl
from jax.experimental.pallas import tpu as pltpu
from jax.experimental.pallas import tpu_sc as plsc

_VOCAB = 100000
_DIM = 128
_BATCH = 4096
_SEQ = 50
_B = _BATCH * _SEQ
_NC = 2
_NS = 16
_NW = _NC * _NS
_BPW = _B // _NW
_CH = 128
_NCH = _BPW // _CH
_NBUF = 5
_NGRP = _NCH // _NBUF


def _make_emb_kernel():
    mesh = plsc.VectorSubcoreMesh(core_axis_name="c", subcore_axis_name="s")

    @functools.partial(
        pl.kernel,
        mesh=mesh,
        out_type=jax.ShapeDtypeStruct((_B, _DIM), jnp.float32),
        scratch_types=[
            pltpu.VMEM((_NCH, _CH), jnp.int32),
            pltpu.VMEM((_NBUF, _CH, _DIM), jnp.float32),
        ]
        + [pltpu.SemaphoreType.DMA] * (2 * _NBUF),
    )
    def emb(idx_hbm, table_hbm, out_hbm, idx_v, bufs, *sems):
        gs, ws = sems[:_NBUF], sems[_NBUF:]
        wid = lax.axis_index("s") * _NC + lax.axis_index("c")
        base = wid * _BPW
        pltpu.sync_copy(idx_hbm.at[wid], idx_v)

        for b in range(_NBUF):
            pltpu.async_copy(table_hbm.at[idx_v.at[b]], bufs.at[b], gs[b])

        @pl.loop(0, _NGRP - 1)
        def _(g):
            for b in range(_NBUF):
                j = g * _NBUF + b
                pltpu.make_async_copy(
                    table_hbm.at[idx_v.at[b]], bufs.at[b], gs[b]).wait()
                pltpu.async_copy(
                    bufs.at[b], out_hbm.at[pl.ds(base + j * _CH, _CH)], ws[b])
            for b in range(_NBUF):
                j2 = (g + 1) * _NBUF + b
                pltpu.make_async_copy(
                    bufs.at[b], out_hbm.at[pl.ds(base, _CH)], ws[b]).wait()
                pltpu.async_copy(table_hbm.at[idx_v.at[j2]], bufs.at[b], gs[b])

        for b in range(_NBUF):
            j = (_NGRP - 1) * _NBUF + b
            pltpu.make_async_copy(
                table_hbm.at[idx_v.at[b]], bufs.at[b], gs[b]).wait()
            pltpu.async_copy(
                bufs.at[b], out_hbm.at[pl.ds(base + j * _CH, _CH)], ws[b])
        for b in range(_NBUF):
            pltpu.make_async_copy(
                bufs.at[b], out_hbm.at[pl.ds(base, _CH)], ws[b]).wait()

    return emb


_emb = _make_emb_kernel()


def kernel(x_in, weight):
    idx = x_in.astype(jnp.int32).reshape(_NW, _NCH, _CH)
    out = _emb(idx, weight)
    return out.reshape(_BATCH, _SEQ, _DIM)
